# baseline (speedup 1.0000x reference)
import functools

import jax
import jax.numpy as jnp
from jax import lax
from jax.experimental import pallas as pl
from jax.experimental.pallas import tpu as pltpu
from jax.experimental.pallas import tpu_sc as plsc

NC = 2
NS = 16
NW = NC * NS


def _sc_mesh():
    return plsc.VectorSubcoreMesh(
        core_axis_name="c", subcore_axis_name="s", num_cores=NC, num_subcores=NS
    )


def _sc_gather(table, idx, ch=80, nbuf=5):
    rows, h = table.shape
    n = idx.shape[0]
    per = n // NW
    groups = per // (ch * nbuf)
    assert per % (ch * nbuf) == 0 and ch % 8 == 0

    @functools.partial(
        pl.kernel,
        mesh=_sc_mesh(),
        out_type=jax.ShapeDtypeStruct((n, h), jnp.float32),
        scratch_types=[
            pltpu.VMEM((per,), jnp.int32),
            pltpu.VMEM((nbuf, ch, h), jnp.float32),
            pltpu.SemaphoreType.DMA,
            pltpu.SemaphoreType.DMA,
            pltpu.SemaphoreType.DMA,
        ],
    )
    def k(table_hbm, idx_hbm, out_hbm, idx_v, rows_v, isem, gsem, wsem):
        wid = lax.axis_index("s") * NC + lax.axis_index("c")
        base = wid * per
        pltpu.async_copy(idx_hbm.at[pl.ds(base, per)], idx_v, isem).wait()

        @pl.loop(0, groups)
        def _(g):
            j0 = g * (ch * nbuf)
            for b in range(nbuf):
                @pl.when(g > 0)
                def _():
                    pltpu.make_async_copy(
                        rows_v.at[b], out_hbm.at[pl.ds(0, ch)], wsem
                    ).wait()
                pltpu.async_copy(
                    table_hbm.at[idx_v.at[pl.ds(pl.multiple_of(j0 + b * ch, 8), ch)]],
                    rows_v.at[b],
                    gsem,
                )
            for b in range(nbuf):
                pltpu.make_async_copy(
                    table_hbm.at[pl.ds(0, ch)], rows_v.at[b], gsem
                ).wait()
                off = pl.multiple_of(base + j0 + b * ch, 8)
                pltpu.async_copy(rows_v.at[b], out_hbm.at[pl.ds(off, ch)], wsem)

        for b in range(nbuf):
            pltpu.make_async_copy(rows_v.at[b], out_hbm.at[pl.ds(0, ch)], wsem).wait()

    return k(table, idx)


def _sc_scatter_add(payload, idx, zeros, o_pad, ch=40, nbuf=5):
    n, h = payload.shape
    per = n // NW
    groups = per // (ch * nbuf)
    rpt = o_pad // NS
    assert per % (ch * nbuf) == 0 and rpt % 8 == 0

    @functools.partial(
        pl.kernel,
        mesh=_sc_mesh(),
        out_type=jax.ShapeDtypeStruct((NC, o_pad, h), jnp.float32),
        scratch_types=[
            pltpu.VMEM((nbuf, ch), jnp.int32),
            pltpu.VMEM((nbuf, ch, h), jnp.float32),
            pltpu.VMEM_SHARED((o_pad, h), jnp.float32),
            pltpu.SemaphoreType.DMA,
            pltpu.SemaphoreType.DMA,
            pltpu.SemaphoreType.DMA,
        ],
    )
    def k(pay_hbm, idx_hbm, zero_hbm, out_hbm, idx_v, vals_v, acc_sh, isem, psem, ssem):
        cid = lax.axis_index("c")
        sid = lax.axis_index("s")
        wid = sid * NC + cid

        pltpu.sync_copy(
            zero_hbm.at[pl.ds(sid * rpt, rpt)], acc_sh.at[pl.ds(sid * rpt, rpt)]
        )
        plsc.subcore_barrier()

        base = wid * per

        @pl.loop(0, groups)
        def _(g):
            j0 = g * (ch * nbuf)
            for b in range(nbuf):
                @pl.when(g > 0)
                def _():
                    pltpu.make_async_copy(
                        vals_v.at[b], acc_sh.at[pl.ds(0, ch)], ssem
                    ).wait()
                off = pl.multiple_of(base + j0 + b * ch, 8)
                pltpu.async_copy(idx_hbm.at[pl.ds(off, ch)], idx_v.at[b], isem)
                pltpu.async_copy(pay_hbm.at[pl.ds(off, ch)], vals_v.at[b], psem)
            for b in range(nbuf):
                pltpu.make_async_copy(
                    idx_hbm.at[pl.ds(0, ch)], idx_v.at[b], isem
                ).wait()
                pltpu.make_async_copy(
                    pay_hbm.at[pl.ds(0, ch)], vals_v.at[b], psem
                ).wait()
                pltpu.async_copy(vals_v.at[b], acc_sh.at[idx_v.at[b]], ssem, add=True)

        for b in range(nbuf):
            pltpu.make_async_copy(vals_v.at[b], acc_sh.at[pl.ds(0, ch)], ssem).wait()

        plsc.subcore_barrier()
        pltpu.sync_copy(
            acc_sh.at[pl.ds(sid * rpt, rpt)],
            out_hbm.at[cid, pl.ds(sid * rpt, rpt)],
        )

    return k(payload, idx, zeros)


def _sc_counts(idx, zeros, ones, o_pad, ch=80, nbuf=5, w=128):
    n = idx.shape[0]
    per = n // NW
    groups = per // (ch * nbuf)
    rpt = o_pad // NS
    assert per % (ch * nbuf) == 0 and rpt % 8 == 0

    @functools.partial(
        pl.kernel,
        mesh=_sc_mesh(),
        out_type=jax.ShapeDtypeStruct((NC, o_pad, w), jnp.float32),
        scratch_types=[
            pltpu.VMEM((nbuf, ch), jnp.int32),
            pltpu.VMEM((ch, w), jnp.float32),
            pltpu.VMEM_SHARED((o_pad, w), jnp.float32),
            pltpu.SemaphoreType.DMA,
            pltpu.SemaphoreType.DMA,
        ],
    )
    def k(idx_hbm, zero_hbm, ones_hbm, out_hbm, idx_v, ones_v, acc_sh, isem, ssem):
        cid = lax.axis_index("c")
        sid = lax.axis_index("s")
        wid = sid * NC + cid

        pltpu.sync_copy(ones_hbm, ones_v)
        pltpu.sync_copy(
            zero_hbm.at[pl.ds(sid * rpt, rpt)], acc_sh.at[pl.ds(sid * rpt, rpt)]
        )
        plsc.subcore_barrier()

        base = wid * per

        @pl.loop(0, groups)
        def _(g):
            j0 = g * (ch * nbuf)
            for b in range(nbuf):
                @pl.when(g > 0)
                def _():
                    pltpu.make_async_copy(
                        ones_v, acc_sh.at[pl.ds(0, ch)], ssem
                    ).wait()
                off = pl.multiple_of(base + j0 + b * ch, 8)
                pltpu.async_copy(idx_hbm.at[pl.ds(off, ch)], idx_v.at[b], isem)
            for b in range(nbuf):
                pltpu.make_async_copy(
                    idx_hbm.at[pl.ds(0, ch)], idx_v.at[b], isem
                ).wait()
                pltpu.async_copy(ones_v, acc_sh.at[idx_v.at[b]], ssem, add=True)

        for b in range(nbuf):
            pltpu.make_async_copy(ones_v, acc_sh.at[pl.ds(0, ch)], ssem).wait()

        plsc.subcore_barrier()
        pltpu.sync_copy(
            acc_sh.at[pl.ds(sid * rpt, rpt)],
            out_hbm.at[cid, pl.ds(sid * rpt, rpt)],
        )

    return k(idx, zeros, ones)


def _embed_rows(ids2, emb, obs=2000):
    n = ids2.shape[0]
    v, d = emb.shape

    def body(id_ref, emb_ref, out_ref):
        oh = (id_ref[...] == lax.broadcasted_iota(jnp.int32, (obs, v), 1)).astype(
            jnp.float32
        )
        out_ref[...] = jnp.dot(oh, emb_ref[...], preferred_element_type=jnp.float32, precision=lax.Precision.HIGHEST)

    return pl.pallas_call(
        body,
        grid=(n // obs,),
        in_specs=[
            pl.BlockSpec((obs, 1), lambda i: (i, 0)),
            pl.BlockSpec((v, d), lambda i: (0, 0)),
        ],
        out_specs=pl.BlockSpec((obs, d), lambda i: (i, 0)),
        out_shape=jax.ShapeDtypeStruct((n, d), jnp.float32),
    )(ids2, emb)


def _node_proj(ov, w1s, w1o, obs=2000):
    o, din = ov.shape
    h = w1s.shape[1]

    def body(ov_ref, ws_ref, wo_ref, out_ref):
        x = ov_ref[...]
        out_ref[0] = jnp.dot(x, ws_ref[...], preferred_element_type=jnp.float32, precision=lax.Precision.HIGHEST)
        out_ref[1] = jnp.dot(x, wo_ref[...], preferred_element_type=jnp.float32, precision=lax.Precision.HIGHEST)

    c = pl.pallas_call(
        body,
        grid=(o // obs,),
        in_specs=[
            pl.BlockSpec((obs, din), lambda i: (i, 0)),
            pl.BlockSpec((din, h), lambda i: (0, 0)),
            pl.BlockSpec((din, h), lambda i: (0, 0)),
        ],
        out_specs=pl.BlockSpec((2, obs, h), lambda i: (0, i, 0)),
        out_shape=jax.ShapeDtypeStruct((2, o, h), jnp.float32),
    )(ov, w1s, w1o)
    return c.reshape(2 * o, h)


def _edge_mlp(g3, pred, w1p, b1, w2, b2, ebs=2000):
    t_edges, h = g3.shape[1], g3.shape[2]
    din = pred.shape[1]
    d2 = w2.shape[1]
    d = d2 - 2 * h

    def body(g_ref, p_ref, wp_ref, b1_ref, w2_ref, b2_ref, np_ref, pay_ref):
        g = g_ref[0] + g_ref[1]
        hh = jnp.maximum(
            g
            + jnp.dot(p_ref[...], wp_ref[...], preferred_element_type=jnp.float32, precision=lax.Precision.HIGHEST)
            + b1_ref[...],
            0.0,
        )
        t = jnp.maximum(
            jnp.dot(hh, w2_ref[...], preferred_element_type=jnp.float32, precision=lax.Precision.HIGHEST) + b2_ref[...],
            0.0,
        )
        np_ref[...] = t[:, h : h + d]
        pay_ref[0] = t[:, :h]
        pay_ref[1] = t[:, h + d :]

    return pl.pallas_call(
        body,
        grid=(t_edges // ebs,),
        in_specs=[
            pl.BlockSpec((2, ebs, h), lambda i: (0, i, 0)),
            pl.BlockSpec((ebs, din), lambda i: (i, 0)),
            pl.BlockSpec((din, h), lambda i: (0, 0)),
            pl.BlockSpec((1, h), lambda i: (0, 0)),
            pl.BlockSpec((h, d2), lambda i: (0, 0)),
            pl.BlockSpec((1, d2), lambda i: (0, 0)),
        ],
        out_specs=[
            pl.BlockSpec((ebs, d), lambda i: (i, 0)),
            pl.BlockSpec((2, ebs, h), lambda i: (0, i, 0)),
        ],
        out_shape=[
            jax.ShapeDtypeStruct((t_edges, d), jnp.float32),
            jax.ShapeDtypeStruct((2, t_edges, h), jnp.float32),
        ],
    )(g3, pred, w1p, b1, w2, b2)


def _edge_mlp0(g3, p2, pred_emb, w1p, b1, w2, b2, ebs=2000):
    t_edges, h = g3.shape[1], g3.shape[2]
    npred, _ = pred_emb.shape
    d2 = w2.shape[1]
    d = d2 - 2 * h

    def body(g_ref, p_ref, pe_ref, wp_ref, b1_ref, w2_ref, b2_ref, np_ref, pay_ref):
        pp = jnp.dot(pe_ref[...], wp_ref[...], preferred_element_type=jnp.float32, precision=lax.Precision.HIGHEST)
        oh = (p_ref[...] == lax.broadcasted_iota(jnp.int32, (ebs, npred), 1)).astype(
            jnp.float32
        )
        g = g_ref[0] + g_ref[1]
        hh = jnp.maximum(
            g + jnp.dot(oh, pp, preferred_element_type=jnp.float32, precision=lax.Precision.HIGHEST) + b1_ref[...], 0.0
        )
        t = jnp.maximum(
            jnp.dot(hh, w2_ref[...], preferred_element_type=jnp.float32, precision=lax.Precision.HIGHEST) + b2_ref[...],
            0.0,
        )
        np_ref[...] = t[:, h : h + d]
        pay_ref[0] = t[:, :h]
        pay_ref[1] = t[:, h + d :]

    din = pred_emb.shape[1]
    return pl.pallas_call(
        body,
        grid=(t_edges // ebs,),
        in_specs=[
            pl.BlockSpec((2, ebs, h), lambda i: (0, i, 0)),
            pl.BlockSpec((ebs, 1), lambda i: (i, 0)),
            pl.BlockSpec((npred, din), lambda i: (0, 0)),
            pl.BlockSpec((din, h), lambda i: (0, 0)),
            pl.BlockSpec((1, h), lambda i: (0, 0)),
            pl.BlockSpec((h, d2), lambda i: (0, 0)),
            pl.BlockSpec((1, d2), lambda i: (0, 0)),
        ],
        out_specs=[
            pl.BlockSpec((ebs, d), lambda i: (i, 0)),
            pl.BlockSpec((2, ebs, h), lambda i: (0, i, 0)),
        ],
        out_shape=[
            jax.ShapeDtypeStruct((t_edges, d), jnp.float32),
            jax.ShapeDtypeStruct((2, t_edges, h), jnp.float32),
        ],
    )(g3, p2, pred_emb, w1p, b1, w2, b2)


def _counts_to_rinv(cparts, o_nodes, obs=2000, w=128):
    o = o_nodes

    def body(c_ref, out_ref):
        c = jnp.sum(c_ref[0] + c_ref[1], axis=1, keepdims=True) * (1.0 / w)
        out_ref[...] = 1.0 / jnp.maximum(c, 1.0)

    return pl.pallas_call(
        body,
        grid=(o // obs,),
        in_specs=[pl.BlockSpec((NC, obs, w), lambda i: (0, i, 0))],
        out_specs=pl.BlockSpec((obs, 1), lambda i: (i, 0)),
        out_shape=jax.ShapeDtypeStruct((o, 1), jnp.float32),
    )(cparts)


def _node_update(parts_a, parts_b, rinv, w3, b3, w4, b4, o_nodes, obs=2000):
    o, h = o_nodes, parts_a.shape[2]
    d = w4.shape[1]

    def body(pa_ref, pb_ref, ri_ref, w3_ref, b3_ref, w4_ref, b4_ref, out_ref):
        pooled = ((pa_ref[0] + pa_ref[1]) + (pb_ref[0] + pb_ref[1])) * ri_ref[...]
        x = jnp.maximum(
            jnp.dot(pooled, w3_ref[...], preferred_element_type=jnp.float32, precision=lax.Precision.HIGHEST)
            + b3_ref[...],
            0.0,
        )
        out_ref[...] = jnp.maximum(
            jnp.dot(x, w4_ref[...], preferred_element_type=jnp.float32, precision=lax.Precision.HIGHEST) + b4_ref[...],
            0.0,
        )

    return pl.pallas_call(
        body,
        grid=(o // obs,),
        in_specs=[
            pl.BlockSpec((NC, obs, h), lambda i: (0, i, 0)),
            pl.BlockSpec((NC, obs, h), lambda i: (0, i, 0)),
            pl.BlockSpec((obs, 1), lambda i: (i, 0)),
            pl.BlockSpec((h, h), lambda i: (0, 0)),
            pl.BlockSpec((1, h), lambda i: (0, 0)),
            pl.BlockSpec((h, d), lambda i: (0, 0)),
            pl.BlockSpec((1, d), lambda i: (0, 0)),
        ],
        out_specs=pl.BlockSpec((obs, d), lambda i: (i, 0)),
        out_shape=jax.ShapeDtypeStruct((o, d), jnp.float32),
    )(parts_a, parts_b, rinv, w3, b3, w4, b4)


def kernel(objs, triples, obj_emb, pred_emb, gconv_params):
    objs = objs.astype(jnp.int32)
    s_idx = triples[:, 0].astype(jnp.int32)
    p_idx = triples[:, 1].astype(jnp.int32)
    o_idx = triples[:, 2].astype(jnp.int32)
    o_nodes = objs.shape[0]
    t_edges = s_idx.shape[0]
    obj_emb = obj_emb.astype(jnp.float32)
    pred_emb = pred_emb.astype(jnp.float32)

    th = t_edges // 2
    halves = []
    for sl in (slice(0, th), slice(th, t_edges)):
        s_h, o_h, p_h = s_idx[sl], o_idx[sl], p_idx[sl]
        halves.append(
            dict(
                cat=jnp.concatenate([s_h, o_h + o_nodes]),
                scat=jnp.concatenate([s_h, o_h]),
                p2=p_h.reshape(th, 1),
            )
        )
    scat_idx = jnp.concatenate([s_idx, o_idx])

    o_pad = -(-o_nodes // (NS * 128)) * (NS * 128)
    ones_w = jnp.ones((80, 128), jnp.float32)
    zeros_h = jnp.zeros((o_pad, 128), jnp.float32)
    obj_vecs = _embed_rows(objs.reshape(o_nodes, 1), obj_emb)
    cparts = _sc_counts(scat_idx, zeros_h, ones_w, o_pad)
    rinv = _counts_to_rinv(cparts, o_nodes)

    pred_vecs = [None, None]
    for li, (w1, b1, w2, b2, w3, b3, w4, b4) in enumerate(gconv_params):
        din = w1.shape[0] // 3
        h = w2.shape[0]
        w1s, w1p, w1o = w1[:din], w1[din : 2 * din], w1[2 * din :]
        ctab = _node_proj(obj_vecs, w1s, w1o)
        g3s = [_sc_gather(ctab, hv["cat"], ch=40).reshape(2, th, h) for hv in halves]
        new_p, pay = [None, None], [None, None]
        for x in range(2):
            if li == 0:
                new_p[x], pay[x] = _edge_mlp0(
                    g3s[x], halves[x]["p2"], pred_emb, w1p,
                    b1.reshape(1, -1), w2, b2.reshape(1, -1),
                )
            else:
                new_p[x], pay[x] = _edge_mlp(
                    g3s[x], pred_vecs[x], w1p, b1.reshape(1, -1), w2, b2.reshape(1, -1)
                )
        parts = [
            _sc_scatter_add(pay[x].reshape(2 * th, h), halves[x]["scat"], zeros_h, o_pad)
            for x in range(2)
        ]
        obj_vecs = _node_update(
            parts[0], parts[1], rinv, w3, b3.reshape(1, -1), w4, b4.reshape(1, -1),
            o_nodes,
        )
        pred_vecs = new_p
    return obj_vecs, jnp.concatenate(pred_vecs, axis=0)


# speedup vs baseline: 1.3860x; 1.3860x over previous
import functools

import jax
import jax.numpy as jnp
from jax import lax
from jax.experimental import pallas as pl
from jax.experimental.pallas import tpu as pltpu
from jax.experimental.pallas import tpu_sc as plsc

NC = 2
NS = 16
NW = NC * NS


def _sc_mesh():
    return plsc.VectorSubcoreMesh(
        core_axis_name="c", subcore_axis_name="s", num_cores=NC, num_subcores=NS
    )


def _sc_gather(table, idx, ch=80, nbuf=5):
    rows, h = table.shape
    n = idx.shape[0]
    per = n // NW
    groups = per // (ch * nbuf)
    assert per % (ch * nbuf) == 0 and ch % 8 == 0

    @functools.partial(
        pl.kernel,
        mesh=_sc_mesh(),
        out_type=jax.ShapeDtypeStruct((n, h), jnp.float32),
        scratch_types=[
            pltpu.VMEM((per,), jnp.int32),
            pltpu.VMEM((nbuf, ch, h), jnp.float32),
            pltpu.SemaphoreType.DMA,
            pltpu.SemaphoreType.DMA,
            pltpu.SemaphoreType.DMA,
        ],
    )
    def k(table_hbm, idx_hbm, out_hbm, idx_v, rows_v, isem, gsem, wsem):
        wid = lax.axis_index("s") * NC + lax.axis_index("c")
        base = wid * per
        pltpu.async_copy(idx_hbm.at[pl.ds(base, per)], idx_v, isem).wait()

        @pl.loop(0, groups)
        def _(g):
            j0 = g * (ch * nbuf)
            for b in range(nbuf):
                @pl.when(g > 0)
                def _():
                    pltpu.make_async_copy(
                        rows_v.at[b], out_hbm.at[pl.ds(0, ch)], wsem
                    ).wait()
                pltpu.async_copy(
                    table_hbm.at[idx_v.at[pl.ds(pl.multiple_of(j0 + b * ch, 8), ch)]],
                    rows_v.at[b],
                    gsem,
                )
            for b in range(nbuf):
                pltpu.make_async_copy(
                    table_hbm.at[pl.ds(0, ch)], rows_v.at[b], gsem
                ).wait()
                off = pl.multiple_of(base + j0 + b * ch, 8)
                pltpu.async_copy(rows_v.at[b], out_hbm.at[pl.ds(off, ch)], wsem)

        for b in range(nbuf):
            pltpu.make_async_copy(rows_v.at[b], out_hbm.at[pl.ds(0, ch)], wsem).wait()

    return k(table, idx)


def _sc_scatter_add(payload, idx, zeros, o_pad, ch=40, nbuf=5):
    n, h = payload.shape
    per = n // NW
    groups = per // (ch * nbuf)
    rpt = o_pad // NS
    assert per % (ch * nbuf) == 0 and rpt % 8 == 0

    @functools.partial(
        pl.kernel,
        mesh=_sc_mesh(),
        out_type=jax.ShapeDtypeStruct((NC, o_pad, h), jnp.float32),
        scratch_types=[
            pltpu.VMEM((nbuf, ch), jnp.int32),
            pltpu.VMEM((nbuf, ch, h), jnp.float32),
            pltpu.VMEM_SHARED((o_pad, h), jnp.float32),
            pltpu.SemaphoreType.DMA,
            pltpu.SemaphoreType.DMA,
            pltpu.SemaphoreType.DMA,
        ],
    )
    def k(pay_hbm, idx_hbm, zero_hbm, out_hbm, idx_v, vals_v, acc_sh, isem, psem, ssem):
        cid = lax.axis_index("c")
        sid = lax.axis_index("s")
        wid = sid * NC + cid

        pltpu.sync_copy(
            zero_hbm.at[pl.ds(sid * rpt, rpt)], acc_sh.at[pl.ds(sid * rpt, rpt)]
        )
        plsc.subcore_barrier()

        base = wid * per

        @pl.loop(0, groups)
        def _(g):
            j0 = g * (ch * nbuf)
            for b in range(nbuf):
                @pl.when(g > 0)
                def _():
                    pltpu.make_async_copy(
                        vals_v.at[b], acc_sh.at[pl.ds(0, ch)], ssem
                    ).wait()
                off = pl.multiple_of(base + j0 + b * ch, 8)
                pltpu.async_copy(idx_hbm.at[pl.ds(off, ch)], idx_v.at[b], isem)
                pltpu.async_copy(pay_hbm.at[pl.ds(off, ch)], vals_v.at[b], psem)
            for b in range(nbuf):
                pltpu.make_async_copy(
                    idx_hbm.at[pl.ds(0, ch)], idx_v.at[b], isem
                ).wait()
                pltpu.make_async_copy(
                    pay_hbm.at[pl.ds(0, ch)], vals_v.at[b], psem
                ).wait()
                pltpu.async_copy(vals_v.at[b], acc_sh.at[idx_v.at[b]], ssem, add=True)

        for b in range(nbuf):
            pltpu.make_async_copy(vals_v.at[b], acc_sh.at[pl.ds(0, ch)], ssem).wait()

        plsc.subcore_barrier()
        pltpu.sync_copy(
            acc_sh.at[pl.ds(sid * rpt, rpt)],
            out_hbm.at[cid, pl.ds(sid * rpt, rpt)],
        )

    return k(payload, idx, zeros)


def _sc_counts(idx, zeros, ones, o_pad, ch=80, nbuf=5, w=128):
    n = idx.shape[0]
    per = n // NW
    groups = per // (ch * nbuf)
    rpt = o_pad // NS
    assert per % (ch * nbuf) == 0 and rpt % 8 == 0

    @functools.partial(
        pl.kernel,
        mesh=_sc_mesh(),
        out_type=jax.ShapeDtypeStruct((NC, o_pad, w), jnp.float32),
        scratch_types=[
            pltpu.VMEM((nbuf, ch), jnp.int32),
            pltpu.VMEM((ch, w), jnp.float32),
            pltpu.VMEM_SHARED((o_pad, w), jnp.float32),
            pltpu.SemaphoreType.DMA,
            pltpu.SemaphoreType.DMA,
        ],
    )
    def k(idx_hbm, zero_hbm, ones_hbm, out_hbm, idx_v, ones_v, acc_sh, isem, ssem):
        cid = lax.axis_index("c")
        sid = lax.axis_index("s")
        wid = sid * NC + cid

        pltpu.sync_copy(ones_hbm, ones_v)
        pltpu.sync_copy(
            zero_hbm.at[pl.ds(sid * rpt, rpt)], acc_sh.at[pl.ds(sid * rpt, rpt)]
        )
        plsc.subcore_barrier()

        base = wid * per

        @pl.loop(0, groups)
        def _(g):
            j0 = g * (ch * nbuf)
            for b in range(nbuf):
                @pl.when(g > 0)
                def _():
                    pltpu.make_async_copy(
                        ones_v, acc_sh.at[pl.ds(0, ch)], ssem
                    ).wait()
                off = pl.multiple_of(base + j0 + b * ch, 8)
                pltpu.async_copy(idx_hbm.at[pl.ds(off, ch)], idx_v.at[b], isem)
            for b in range(nbuf):
                pltpu.make_async_copy(
                    idx_hbm.at[pl.ds(0, ch)], idx_v.at[b], isem
                ).wait()
                pltpu.async_copy(ones_v, acc_sh.at[idx_v.at[b]], ssem, add=True)

        for b in range(nbuf):
            pltpu.make_async_copy(ones_v, acc_sh.at[pl.ds(0, ch)], ssem).wait()

        plsc.subcore_barrier()
        pltpu.sync_copy(
            acc_sh.at[pl.ds(sid * rpt, rpt)],
            out_hbm.at[cid, pl.ds(sid * rpt, rpt)],
        )

    return k(idx, zeros, ones)


def _embed_rows(ids2, emb, obs=2000):
    n = ids2.shape[0]
    v, d = emb.shape

    def body(id_ref, emb_ref, out_ref):
        oh = (id_ref[...] == lax.broadcasted_iota(jnp.int32, (obs, v), 1)).astype(
            jnp.float32
        )
        out_ref[...] = jnp.dot(oh, emb_ref[...], preferred_element_type=jnp.float32)

    return pl.pallas_call(
        body,
        grid=(n // obs,),
        in_specs=[
            pl.BlockSpec((obs, 1), lambda i: (i, 0)),
            pl.BlockSpec((v, d), lambda i: (0, 0)),
        ],
        out_specs=pl.BlockSpec((obs, d), lambda i: (i, 0)),
        out_shape=jax.ShapeDtypeStruct((n, d), jnp.float32),
    )(ids2, emb)


def _node_proj(ov, w1s, w1o, obs=2000):
    o, din = ov.shape
    h = w1s.shape[1]

    def body(ov_ref, ws_ref, wo_ref, out_ref):
        x = ov_ref[...]
        out_ref[0] = jnp.dot(x, ws_ref[...], preferred_element_type=jnp.float32)
        out_ref[1] = jnp.dot(x, wo_ref[...], preferred_element_type=jnp.float32)

    c = pl.pallas_call(
        body,
        grid=(o // obs,),
        in_specs=[
            pl.BlockSpec((obs, din), lambda i: (i, 0)),
            pl.BlockSpec((din, h), lambda i: (0, 0)),
            pl.BlockSpec((din, h), lambda i: (0, 0)),
        ],
        out_specs=pl.BlockSpec((2, obs, h), lambda i: (0, i, 0)),
        out_shape=jax.ShapeDtypeStruct((2, o, h), jnp.float32),
    )(ov, w1s, w1o)
    return c.reshape(2 * o, h)


def _edge_mlp(g3, pred, w1p, b1, w2, b2, ebs=2000):
    t_edges, h = g3.shape[1], g3.shape[2]
    din = pred.shape[1]
    d2 = w2.shape[1]
    d = d2 - 2 * h

    def body(g_ref, p_ref, wp_ref, b1_ref, w2_ref, b2_ref, np_ref, pay_ref):
        g = g_ref[0] + g_ref[1]
        hh = jnp.maximum(
            g
            + jnp.dot(p_ref[...], wp_ref[...], preferred_element_type=jnp.float32)
            + b1_ref[...],
            0.0,
        )
        t = jnp.maximum(
            jnp.dot(hh, w2_ref[...], preferred_element_type=jnp.float32) + b2_ref[...],
            0.0,
        )
        np_ref[...] = t[:, h : h + d]
        pay_ref[0] = t[:, :h]
        pay_ref[1] = t[:, h + d :]

    return pl.pallas_call(
        body,
        grid=(t_edges // ebs,),
        in_specs=[
            pl.BlockSpec((2, ebs, h), lambda i: (0, i, 0)),
            pl.BlockSpec((ebs, din), lambda i: (i, 0)),
            pl.BlockSpec((din, h), lambda i: (0, 0)),
            pl.BlockSpec((1, h), lambda i: (0, 0)),
            pl.BlockSpec((h, d2), lambda i: (0, 0)),
            pl.BlockSpec((1, d2), lambda i: (0, 0)),
        ],
        out_specs=[
            pl.BlockSpec((ebs, d), lambda i: (i, 0)),
            pl.BlockSpec((2, ebs, h), lambda i: (0, i, 0)),
        ],
        out_shape=[
            jax.ShapeDtypeStruct((t_edges, d), jnp.float32),
            jax.ShapeDtypeStruct((2, t_edges, h), jnp.float32),
        ],
    )(g3, pred, w1p, b1, w2, b2)


def _edge_mlp0(g3, p2, pred_emb, w1p, b1, w2, b2, ebs=2000):
    t_edges, h = g3.shape[1], g3.shape[2]
    npred, _ = pred_emb.shape
    d2 = w2.shape[1]
    d = d2 - 2 * h

    def body(g_ref, p_ref, pe_ref, wp_ref, b1_ref, w2_ref, b2_ref, np_ref, pay_ref):
        pp = jnp.dot(pe_ref[...], wp_ref[...], preferred_element_type=jnp.float32)
        oh = (p_ref[...] == lax.broadcasted_iota(jnp.int32, (ebs, npred), 1)).astype(
            jnp.float32
        )
        g = g_ref[0] + g_ref[1]
        hh = jnp.maximum(
            g + jnp.dot(oh, pp, preferred_element_type=jnp.float32) + b1_ref[...], 0.0
        )
        t = jnp.maximum(
            jnp.dot(hh, w2_ref[...], preferred_element_type=jnp.float32) + b2_ref[...],
            0.0,
        )
        np_ref[...] = t[:, h : h + d]
        pay_ref[0] = t[:, :h]
        pay_ref[1] = t[:, h + d :]

    din = pred_emb.shape[1]
    return pl.pallas_call(
        body,
        grid=(t_edges // ebs,),
        in_specs=[
            pl.BlockSpec((2, ebs, h), lambda i: (0, i, 0)),
            pl.BlockSpec((ebs, 1), lambda i: (i, 0)),
            pl.BlockSpec((npred, din), lambda i: (0, 0)),
            pl.BlockSpec((din, h), lambda i: (0, 0)),
            pl.BlockSpec((1, h), lambda i: (0, 0)),
            pl.BlockSpec((h, d2), lambda i: (0, 0)),
            pl.BlockSpec((1, d2), lambda i: (0, 0)),
        ],
        out_specs=[
            pl.BlockSpec((ebs, d), lambda i: (i, 0)),
            pl.BlockSpec((2, ebs, h), lambda i: (0, i, 0)),
        ],
        out_shape=[
            jax.ShapeDtypeStruct((t_edges, d), jnp.float32),
            jax.ShapeDtypeStruct((2, t_edges, h), jnp.float32),
        ],
    )(g3, p2, pred_emb, w1p, b1, w2, b2)


def _counts_to_rinv(cparts, o_nodes, obs=2000, w=128):
    o = o_nodes

    def body(c_ref, out_ref):
        c = jnp.sum(c_ref[0] + c_ref[1], axis=1, keepdims=True) * (1.0 / w)
        out_ref[...] = 1.0 / jnp.maximum(c, 1.0)

    return pl.pallas_call(
        body,
        grid=(o // obs,),
        in_specs=[pl.BlockSpec((NC, obs, w), lambda i: (0, i, 0))],
        out_specs=pl.BlockSpec((obs, 1), lambda i: (i, 0)),
        out_shape=jax.ShapeDtypeStruct((o, 1), jnp.float32),
    )(cparts)


def _node_update(parts_a, parts_b, rinv, w3, b3, w4, b4, o_nodes, obs=2000):
    o, h = o_nodes, parts_a.shape[2]
    d = w4.shape[1]

    def body(pa_ref, pb_ref, ri_ref, w3_ref, b3_ref, w4_ref, b4_ref, out_ref):
        pooled = ((pa_ref[0] + pa_ref[1]) + (pb_ref[0] + pb_ref[1])) * ri_ref[...]
        x = jnp.maximum(
            jnp.dot(pooled, w3_ref[...], preferred_element_type=jnp.float32)
            + b3_ref[...],
            0.0,
        )
        out_ref[...] = jnp.maximum(
            jnp.dot(x, w4_ref[...], preferred_element_type=jnp.float32) + b4_ref[...],
            0.0,
        )

    return pl.pallas_call(
        body,
        grid=(o // obs,),
        in_specs=[
            pl.BlockSpec((NC, obs, h), lambda i: (0, i, 0)),
            pl.BlockSpec((NC, obs, h), lambda i: (0, i, 0)),
            pl.BlockSpec((obs, 1), lambda i: (i, 0)),
            pl.BlockSpec((h, h), lambda i: (0, 0)),
            pl.BlockSpec((1, h), lambda i: (0, 0)),
            pl.BlockSpec((h, d), lambda i: (0, 0)),
            pl.BlockSpec((1, d), lambda i: (0, 0)),
        ],
        out_specs=pl.BlockSpec((obs, d), lambda i: (i, 0)),
        out_shape=jax.ShapeDtypeStruct((o, d), jnp.float32),
    )(parts_a, parts_b, rinv, w3, b3, w4, b4)


def kernel(objs, triples, obj_emb, pred_emb, gconv_params):
    objs = objs.astype(jnp.int32)
    s_idx = triples[:, 0].astype(jnp.int32)
    p_idx = triples[:, 1].astype(jnp.int32)
    o_idx = triples[:, 2].astype(jnp.int32)
    o_nodes = objs.shape[0]
    t_edges = s_idx.shape[0]
    obj_emb = obj_emb.astype(jnp.float32)
    pred_emb = pred_emb.astype(jnp.float32)

    th = t_edges // 2
    halves = []
    for sl in (slice(0, th), slice(th, t_edges)):
        s_h, o_h, p_h = s_idx[sl], o_idx[sl], p_idx[sl]
        halves.append(
            dict(
                cat=jnp.concatenate([s_h, o_h + o_nodes]),
                scat=jnp.concatenate([s_h, o_h]),
                p2=p_h.reshape(th, 1),
            )
        )
    scat_idx = jnp.concatenate([s_idx, o_idx])

    o_pad = -(-o_nodes // (NS * 128)) * (NS * 128)
    ones_w = jnp.ones((80, 128), jnp.float32)
    zeros_h = jnp.zeros((o_pad, 128), jnp.float32)
    obj_vecs = _embed_rows(objs.reshape(o_nodes, 1), obj_emb)
    cparts = _sc_counts(scat_idx, zeros_h, ones_w, o_pad)
    rinv = _counts_to_rinv(cparts, o_nodes)

    pred_vecs = [None, None]
    for li, (w1, b1, w2, b2, w3, b3, w4, b4) in enumerate(gconv_params):
        din = w1.shape[0] // 3
        h = w2.shape[0]
        w1s, w1p, w1o = w1[:din], w1[din : 2 * din], w1[2 * din :]
        ctab = _node_proj(obj_vecs, w1s, w1o)
        g3s = [_sc_gather(ctab, hv["cat"], ch=40).reshape(2, th, h) for hv in halves]
        new_p, pay = [None, None], [None, None]
        for x in range(2):
            if li == 0:
                new_p[x], pay[x] = _edge_mlp0(
                    g3s[x], halves[x]["p2"], pred_emb, w1p,
                    b1.reshape(1, -1), w2, b2.reshape(1, -1),
                )
            else:
                new_p[x], pay[x] = _edge_mlp(
                    g3s[x], pred_vecs[x], w1p, b1.reshape(1, -1), w2, b2.reshape(1, -1)
                )
        parts = [
            _sc_scatter_add(pay[x].reshape(2 * th, h), halves[x]["scat"], zeros_h, o_pad)
            for x in range(2)
        ]
        obj_vecs = _node_update(
            parts[0], parts[1], rinv, w3, b3.reshape(1, -1), w4, b4.reshape(1, -1),
            o_nodes,
        )
        pred_vecs = new_p
    return obj_vecs, jnp.concatenate(pred_vecs, axis=0)


# speedup vs baseline: 1.4174x; 1.0227x over previous
import functools

import jax
import jax.numpy as jnp
from jax import lax
from jax.experimental import pallas as pl
from jax.experimental.pallas import tpu as pltpu
from jax.experimental.pallas import tpu_sc as plsc

NC = 2
NS = 16
NW = NC * NS


def _sc_mesh():
    return plsc.VectorSubcoreMesh(
        core_axis_name="c", subcore_axis_name="s", num_cores=NC, num_subcores=NS
    )


def _sc_gather(table, idx, ch=80, nbuf=5):
    rows, h = table.shape
    n = idx.shape[0]
    per = n // NW
    groups = per // (ch * nbuf)
    assert per % (ch * nbuf) == 0 and ch % 8 == 0

    @functools.partial(
        pl.kernel,
        mesh=_sc_mesh(),
        out_type=jax.ShapeDtypeStruct((n, h), jnp.float32),
        scratch_types=[
            pltpu.VMEM((per,), jnp.int32),
            pltpu.VMEM((nbuf, ch, h), jnp.float32),
            pltpu.SemaphoreType.DMA,
            pltpu.SemaphoreType.DMA,
            pltpu.SemaphoreType.DMA,
        ],
    )
    def k(table_hbm, idx_hbm, out_hbm, idx_v, rows_v, isem, gsem, wsem):
        wid = lax.axis_index("s") * NC + lax.axis_index("c")
        base = wid * per
        pltpu.async_copy(idx_hbm.at[pl.ds(base, per)], idx_v, isem).wait()

        @pl.loop(0, groups)
        def _(g):
            j0 = g * (ch * nbuf)
            for b in range(nbuf):
                @pl.when(g > 0)
                def _():
                    pltpu.make_async_copy(
                        rows_v.at[b], out_hbm.at[pl.ds(0, ch)], wsem
                    ).wait()
                pltpu.async_copy(
                    table_hbm.at[idx_v.at[pl.ds(pl.multiple_of(j0 + b * ch, 8), ch)]],
                    rows_v.at[b],
                    gsem,
                )
            for b in range(nbuf):
                pltpu.make_async_copy(
                    table_hbm.at[pl.ds(0, ch)], rows_v.at[b], gsem
                ).wait()
                off = pl.multiple_of(base + j0 + b * ch, 8)
                pltpu.async_copy(rows_v.at[b], out_hbm.at[pl.ds(off, ch)], wsem)

        for b in range(nbuf):
            pltpu.make_async_copy(rows_v.at[b], out_hbm.at[pl.ds(0, ch)], wsem).wait()

    return k(table, idx)


def _sc_scatter_add(payload, idx, zeros, o_pad, ch=40, nbuf=5):
    n, h = payload.shape
    per = n // NW
    groups = per // (ch * nbuf)
    rpt = o_pad // NS
    assert per % (ch * nbuf) == 0 and rpt % 8 == 0

    @functools.partial(
        pl.kernel,
        mesh=_sc_mesh(),
        out_type=jax.ShapeDtypeStruct((NC, o_pad, h), jnp.float32),
        scratch_types=[
            pltpu.VMEM((nbuf, ch), jnp.int32),
            pltpu.VMEM((nbuf, ch, h), jnp.float32),
            pltpu.VMEM_SHARED((o_pad, h), jnp.float32),
            pltpu.SemaphoreType.DMA,
            pltpu.SemaphoreType.DMA,
            pltpu.SemaphoreType.DMA,
        ],
    )
    def k(pay_hbm, idx_hbm, zero_hbm, out_hbm, idx_v, vals_v, acc_sh, isem, psem, ssem):
        cid = lax.axis_index("c")
        sid = lax.axis_index("s")
        wid = sid * NC + cid

        pltpu.sync_copy(
            zero_hbm.at[pl.ds(sid * rpt, rpt)], acc_sh.at[pl.ds(sid * rpt, rpt)]
        )
        plsc.subcore_barrier()

        base = wid * per

        @pl.loop(0, groups)
        def _(g):
            j0 = g * (ch * nbuf)
            for b in range(nbuf):
                @pl.when(g > 0)
                def _():
                    pltpu.make_async_copy(
                        vals_v.at[b], acc_sh.at[pl.ds(0, ch)], ssem
                    ).wait()
                off = pl.multiple_of(base + j0 + b * ch, 8)
                pltpu.async_copy(idx_hbm.at[pl.ds(off, ch)], idx_v.at[b], isem)
                pltpu.async_copy(pay_hbm.at[pl.ds(off, ch)], vals_v.at[b], psem)
            for b in range(nbuf):
                pltpu.make_async_copy(
                    idx_hbm.at[pl.ds(0, ch)], idx_v.at[b], isem
                ).wait()
                pltpu.make_async_copy(
                    pay_hbm.at[pl.ds(0, ch)], vals_v.at[b], psem
                ).wait()
                pltpu.async_copy(vals_v.at[b], acc_sh.at[idx_v.at[b]], ssem, add=True)

        for b in range(nbuf):
            pltpu.make_async_copy(vals_v.at[b], acc_sh.at[pl.ds(0, ch)], ssem).wait()

        plsc.subcore_barrier()
        pltpu.sync_copy(
            acc_sh.at[pl.ds(sid * rpt, rpt)],
            out_hbm.at[cid, pl.ds(sid * rpt, rpt)],
        )

    return k(payload, idx, zeros)


def _sc_counts(idx, zeros, ones, o_pad, ch=80, nbuf=5, w=128):
    n = idx.shape[0]
    per = n // NW
    groups = per // (ch * nbuf)
    rpt = o_pad // NS
    assert per % (ch * nbuf) == 0 and rpt % 8 == 0

    @functools.partial(
        pl.kernel,
        mesh=_sc_mesh(),
        out_type=jax.ShapeDtypeStruct((NC, o_pad, w), jnp.float32),
        scratch_types=[
            pltpu.VMEM((nbuf, ch), jnp.int32),
            pltpu.VMEM((ch, w), jnp.float32),
            pltpu.VMEM_SHARED((o_pad, w), jnp.float32),
            pltpu.SemaphoreType.DMA,
            pltpu.SemaphoreType.DMA,
        ],
    )
    def k(idx_hbm, zero_hbm, ones_hbm, out_hbm, idx_v, ones_v, acc_sh, isem, ssem):
        cid = lax.axis_index("c")
        sid = lax.axis_index("s")
        wid = sid * NC + cid

        pltpu.sync_copy(ones_hbm, ones_v)
        pltpu.sync_copy(
            zero_hbm.at[pl.ds(sid * rpt, rpt)], acc_sh.at[pl.ds(sid * rpt, rpt)]
        )
        plsc.subcore_barrier()

        base = wid * per

        @pl.loop(0, groups)
        def _(g):
            j0 = g * (ch * nbuf)
            for b in range(nbuf):
                @pl.when(g > 0)
                def _():
                    pltpu.make_async_copy(
                        ones_v, acc_sh.at[pl.ds(0, ch)], ssem
                    ).wait()
                off = pl.multiple_of(base + j0 + b * ch, 8)
                pltpu.async_copy(idx_hbm.at[pl.ds(off, ch)], idx_v.at[b], isem)
            for b in range(nbuf):
                pltpu.make_async_copy(
                    idx_hbm.at[pl.ds(0, ch)], idx_v.at[b], isem
                ).wait()
                pltpu.async_copy(ones_v, acc_sh.at[idx_v.at[b]], ssem, add=True)

        for b in range(nbuf):
            pltpu.make_async_copy(ones_v, acc_sh.at[pl.ds(0, ch)], ssem).wait()

        plsc.subcore_barrier()
        pltpu.sync_copy(
            acc_sh.at[pl.ds(sid * rpt, rpt)],
            out_hbm.at[cid, pl.ds(sid * rpt, rpt)],
        )

    return k(idx, zeros, ones)


def _embed_proj(ids2, emb, w1s, w1o, obs=2000):
    n = ids2.shape[0]
    v, d = emb.shape
    h = w1s.shape[1]

    def body(id_ref, emb_ref, ws_ref, wo_ref, out_ref):
        oh = (id_ref[...] == lax.broadcasted_iota(jnp.int32, (obs, v), 1)).astype(
            jnp.float32
        )
        e = emb_ref[...]
        ts = jnp.dot(e, ws_ref[...], preferred_element_type=jnp.float32)
        to = jnp.dot(e, wo_ref[...], preferred_element_type=jnp.float32)
        out_ref[0] = jnp.dot(oh, ts, preferred_element_type=jnp.float32)
        out_ref[1] = jnp.dot(oh, to, preferred_element_type=jnp.float32)

    c = pl.pallas_call(
        body,
        grid=(n // obs,),
        in_specs=[
            pl.BlockSpec((obs, 1), lambda i: (i, 0)),
            pl.BlockSpec((v, d), lambda i: (0, 0)),
            pl.BlockSpec((d, h), lambda i: (0, 0)),
            pl.BlockSpec((d, h), lambda i: (0, 0)),
        ],
        out_specs=pl.BlockSpec((2, obs, h), lambda i: (0, i, 0)),
        out_shape=jax.ShapeDtypeStruct((2, n, h), jnp.float32),
    )(ids2, emb, w1s, w1o)
    return c.reshape(2 * n, h)


def _node_proj(ov, w1s, w1o, obs=2000):
    o, din = ov.shape
    h = w1s.shape[1]

    def body(ov_ref, ws_ref, wo_ref, out_ref):
        x = ov_ref[...]
        out_ref[0] = jnp.dot(x, ws_ref[...], preferred_element_type=jnp.float32)
        out_ref[1] = jnp.dot(x, wo_ref[...], preferred_element_type=jnp.float32)

    c = pl.pallas_call(
        body,
        grid=(o // obs,),
        in_specs=[
            pl.BlockSpec((obs, din), lambda i: (i, 0)),
            pl.BlockSpec((din, h), lambda i: (0, 0)),
            pl.BlockSpec((din, h), lambda i: (0, 0)),
        ],
        out_specs=pl.BlockSpec((2, obs, h), lambda i: (0, i, 0)),
        out_shape=jax.ShapeDtypeStruct((2, o, h), jnp.float32),
    )(ov, w1s, w1o)
    return c.reshape(2 * o, h)


def _edge_mlp(g3, pred, w1p, b1, w2, b2, ebs=4000):
    t_edges, h = g3.shape[1], g3.shape[2]
    din = pred.shape[1]
    d2 = w2.shape[1]
    d = d2 - 2 * h

    def body(g_ref, p_ref, wp_ref, b1_ref, w2_ref, b2_ref, np_ref, pay_ref):
        g = g_ref[0] + g_ref[1]
        hh = jnp.maximum(
            g
            + jnp.dot(p_ref[...], wp_ref[...], preferred_element_type=jnp.float32)
            + b1_ref[...],
            0.0,
        )
        t = jnp.maximum(
            jnp.dot(hh, w2_ref[...], preferred_element_type=jnp.float32) + b2_ref[...],
            0.0,
        )
        np_ref[...] = t[:, h : h + d]
        pay_ref[0] = t[:, :h]
        pay_ref[1] = t[:, h + d :]

    return pl.pallas_call(
        body,
        grid=(t_edges // ebs,),
        in_specs=[
            pl.BlockSpec((2, ebs, h), lambda i: (0, i, 0)),
            pl.BlockSpec((ebs, din), lambda i: (i, 0)),
            pl.BlockSpec((din, h), lambda i: (0, 0)),
            pl.BlockSpec((1, h), lambda i: (0, 0)),
            pl.BlockSpec((h, d2), lambda i: (0, 0)),
            pl.BlockSpec((1, d2), lambda i: (0, 0)),
        ],
        out_specs=[
            pl.BlockSpec((ebs, d), lambda i: (i, 0)),
            pl.BlockSpec((2, ebs, h), lambda i: (0, i, 0)),
        ],
        out_shape=[
            jax.ShapeDtypeStruct((t_edges, d), jnp.float32),
            jax.ShapeDtypeStruct((2, t_edges, h), jnp.float32),
        ],
    )(g3, pred, w1p, b1, w2, b2)


def _edge_mlp0(g3, p2, pred_emb, w1p, b1, w2, b2, ebs=4000):
    t_edges, h = g3.shape[1], g3.shape[2]
    npred, _ = pred_emb.shape
    d2 = w2.shape[1]
    d = d2 - 2 * h

    def body(g_ref, p_ref, pe_ref, wp_ref, b1_ref, w2_ref, b2_ref, np_ref, pay_ref):
        pp = jnp.dot(pe_ref[...], wp_ref[...], preferred_element_type=jnp.float32)
        oh = (p_ref[...] == lax.broadcasted_iota(jnp.int32, (ebs, npred), 1)).astype(
            jnp.float32
        )
        g = g_ref[0] + g_ref[1]
        hh = jnp.maximum(
            g + jnp.dot(oh, pp, preferred_element_type=jnp.float32) + b1_ref[...], 0.0
        )
        t = jnp.maximum(
            jnp.dot(hh, w2_ref[...], preferred_element_type=jnp.float32) + b2_ref[...],
            0.0,
        )
        np_ref[...] = t[:, h : h + d]
        pay_ref[0] = t[:, :h]
        pay_ref[1] = t[:, h + d :]

    din = pred_emb.shape[1]
    return pl.pallas_call(
        body,
        grid=(t_edges // ebs,),
        in_specs=[
            pl.BlockSpec((2, ebs, h), lambda i: (0, i, 0)),
            pl.BlockSpec((ebs, 1), lambda i: (i, 0)),
            pl.BlockSpec((npred, din), lambda i: (0, 0)),
            pl.BlockSpec((din, h), lambda i: (0, 0)),
            pl.BlockSpec((1, h), lambda i: (0, 0)),
            pl.BlockSpec((h, d2), lambda i: (0, 0)),
            pl.BlockSpec((1, d2), lambda i: (0, 0)),
        ],
        out_specs=[
            pl.BlockSpec((ebs, d), lambda i: (i, 0)),
            pl.BlockSpec((2, ebs, h), lambda i: (0, i, 0)),
        ],
        out_shape=[
            jax.ShapeDtypeStruct((t_edges, d), jnp.float32),
            jax.ShapeDtypeStruct((2, t_edges, h), jnp.float32),
        ],
    )(g3, p2, pred_emb, w1p, b1, w2, b2)


def _counts_to_rinv(cparts, o_nodes, obs=2000, w=128):
    o = o_nodes

    def body(c_ref, out_ref):
        c = jnp.sum(c_ref[0] + c_ref[1], axis=1, keepdims=True) * (1.0 / w)
        out_ref[...] = 1.0 / jnp.maximum(c, 1.0)

    return pl.pallas_call(
        body,
        grid=(o // obs,),
        in_specs=[pl.BlockSpec((NC, obs, w), lambda i: (0, i, 0))],
        out_specs=pl.BlockSpec((obs, 1), lambda i: (i, 0)),
        out_shape=jax.ShapeDtypeStruct((o, 1), jnp.float32),
    )(cparts)


def _node_update_proj(parts_a, parts_b, rinv, w3, b3, w4, b4, w1s, w1o, o_nodes,
                      obs=2000):
    o, h = o_nodes, parts_a.shape[2]
    d = w4.shape[1]
    hn = w1s.shape[1]

    def body(pa_ref, pb_ref, ri_ref, w3_ref, b3_ref, w4_ref, b4_ref, ws_ref,
             wo_ref, out_ref):
        pooled = ((pa_ref[0] + pa_ref[1]) + (pb_ref[0] + pb_ref[1])) * ri_ref[...]
        x = jnp.maximum(
            jnp.dot(pooled, w3_ref[...], preferred_element_type=jnp.float32)
            + b3_ref[...],
            0.0,
        )
        nv = jnp.maximum(
            jnp.dot(x, w4_ref[...], preferred_element_type=jnp.float32) + b4_ref[...],
            0.0,
        )
        out_ref[0] = jnp.dot(nv, ws_ref[...], preferred_element_type=jnp.float32)
        out_ref[1] = jnp.dot(nv, wo_ref[...], preferred_element_type=jnp.float32)

    c = pl.pallas_call(
        body,
        grid=(o // obs,),
        in_specs=[
            pl.BlockSpec((NC, obs, h), lambda i: (0, i, 0)),
            pl.BlockSpec((NC, obs, h), lambda i: (0, i, 0)),
            pl.BlockSpec((obs, 1), lambda i: (i, 0)),
            pl.BlockSpec((h, h), lambda i: (0, 0)),
            pl.BlockSpec((1, h), lambda i: (0, 0)),
            pl.BlockSpec((h, d), lambda i: (0, 0)),
            pl.BlockSpec((1, d), lambda i: (0, 0)),
            pl.BlockSpec((d, hn), lambda i: (0, 0)),
            pl.BlockSpec((d, hn), lambda i: (0, 0)),
        ],
        out_specs=pl.BlockSpec((2, obs, hn), lambda i: (0, i, 0)),
        out_shape=jax.ShapeDtypeStruct((2, o, hn), jnp.float32),
    )(parts_a, parts_b, rinv, w3, b3, w4, b4, w1s, w1o)
    return c.reshape(2 * o, hn)


def _node_update(parts_a, parts_b, rinv, w3, b3, w4, b4, o_nodes, obs=2000):
    o, h = o_nodes, parts_a.shape[2]
    d = w4.shape[1]

    def body(pa_ref, pb_ref, ri_ref, w3_ref, b3_ref, w4_ref, b4_ref, out_ref):
        pooled = ((pa_ref[0] + pa_ref[1]) + (pb_ref[0] + pb_ref[1])) * ri_ref[...]
        x = jnp.maximum(
            jnp.dot(pooled, w3_ref[...], preferred_element_type=jnp.float32)
            + b3_ref[...],
            0.0,
        )
        out_ref[...] = jnp.maximum(
            jnp.dot(x, w4_ref[...], preferred_element_type=jnp.float32) + b4_ref[...],
            0.0,
        )

    return pl.pallas_call(
        body,
        grid=(o // obs,),
        in_specs=[
            pl.BlockSpec((NC, obs, h), lambda i: (0, i, 0)),
            pl.BlockSpec((NC, obs, h), lambda i: (0, i, 0)),
            pl.BlockSpec((obs, 1), lambda i: (i, 0)),
            pl.BlockSpec((h, h), lambda i: (0, 0)),
            pl.BlockSpec((1, h), lambda i: (0, 0)),
            pl.BlockSpec((h, d), lambda i: (0, 0)),
            pl.BlockSpec((1, d), lambda i: (0, 0)),
        ],
        out_specs=pl.BlockSpec((obs, d), lambda i: (i, 0)),
        out_shape=jax.ShapeDtypeStruct((o, d), jnp.float32),
    )(parts_a, parts_b, rinv, w3, b3, w4, b4)


def kernel(objs, triples, obj_emb, pred_emb, gconv_params):
    objs = objs.astype(jnp.int32)
    s_idx = triples[:, 0].astype(jnp.int32)
    p_idx = triples[:, 1].astype(jnp.int32)
    o_idx = triples[:, 2].astype(jnp.int32)
    o_nodes = objs.shape[0]
    t_edges = s_idx.shape[0]
    obj_emb = obj_emb.astype(jnp.float32)
    pred_emb = pred_emb.astype(jnp.float32)

    th = t_edges // 2
    halves = []
    for sl in (slice(0, th), slice(th, t_edges)):
        s_h, o_h, p_h = s_idx[sl], o_idx[sl], p_idx[sl]
        halves.append(
            dict(
                cat=jnp.concatenate([s_h, o_h + o_nodes]),
                scat=jnp.concatenate([s_h, o_h]),
                p2=p_h.reshape(th, 1),
            )
        )
    scat_idx = jnp.concatenate([s_idx, o_idx])

    o_pad = -(-o_nodes // (NS * 128)) * (NS * 128)
    ones_w = jnp.ones((80, 128), jnp.float32)
    zeros_h = jnp.zeros((o_pad, 128), jnp.float32)
    cparts = _sc_counts(scat_idx, zeros_h, ones_w, o_pad)
    rinv = _counts_to_rinv(cparts, o_nodes)

    n_layers = len(gconv_params)
    pred_vecs = [None, None]
    ctab = None
    obj_vecs = None
    for li, (w1, b1, w2, b2, w3, b3, w4, b4) in enumerate(gconv_params):
        din = w1.shape[0] // 3
        h = w2.shape[0]
        w1p = w1[din : 2 * din]
        if li == 0:
            ctab = _embed_proj(
                objs.reshape(o_nodes, 1), obj_emb, w1[:din], w1[2 * din :]
            )
        g3s = [_sc_gather(ctab, hv["cat"], ch=40).reshape(2, th, h) for hv in halves]
        new_p, pay = [None, None], [None, None]
        for x in range(2):
            if li == 0:
                new_p[x], pay[x] = _edge_mlp0(
                    g3s[x], halves[x]["p2"], pred_emb, w1p,
                    b1.reshape(1, -1), w2, b2.reshape(1, -1),
                )
            else:
                new_p[x], pay[x] = _edge_mlp(
                    g3s[x], pred_vecs[x], w1p, b1.reshape(1, -1), w2, b2.reshape(1, -1)
                )
        parts = [
            _sc_scatter_add(pay[x].reshape(2 * th, h), halves[x]["scat"], zeros_h, o_pad)
            for x in range(2)
        ]
        if li + 1 < n_layers:
            w1n = gconv_params[li + 1][0]
            dn = w1n.shape[0] // 3
            ctab = _node_update_proj(
                parts[0], parts[1], rinv, w3, b3.reshape(1, -1), w4,
                b4.reshape(1, -1), w1n[:dn], w1n[2 * dn :], o_nodes,
            )
        else:
            obj_vecs = _node_update(
                parts[0], parts[1], rinv, w3, b3.reshape(1, -1), w4,
                b4.reshape(1, -1), o_nodes,
            )
        pred_vecs = new_p
    return obj_vecs, jnp.concatenate(pred_vecs, axis=0)
